# baseline (speedup 1.0000x reference)
import functools

import jax
import jax.numpy as jnp
from jax import lax
from jax.experimental import pallas as pl
from jax.experimental.pallas import tpu as pltpu
from jax.experimental.pallas import tpu_sc as plsc

NUM_LABELS = 1000000
EMBED_DIM = 32
BATCH = 16384
HIST = 50
_HH = HIST // 2

_NC = 2
_NS = 16
_NW = _NC * _NS
_ROWS_W = BATCH // _NW


def _make_kernel():
  mesh = plsc.VectorSubcoreMesh(core_axis_name="c", subcore_axis_name="s")

  @functools.partial(
      pl.kernel,
      out_type=jax.ShapeDtypeStruct((_HH, BATCH, EMBED_DIM), jnp.float32),
      mesh=mesh,
      compiler_params=pltpu.CompilerParams(use_tc_tiling_on_sc=False),
      scratch_types=[
          pltpu.VMEM((_HH, _ROWS_W), jnp.int32),
          pltpu.VMEM((_ROWS_W, EMBED_DIM), jnp.float32),
          pltpu.VMEM((_ROWS_W, EMBED_DIM), jnp.float32),
          pltpu.SemaphoreType.DMA,
          pltpu.SemaphoreType.DMA,
      ],
  )
  def gather_kernel(idx_hbm, table_hbm, out_hbm, idx_v, rows0, rows1, g0, g1):
    wid = lax.axis_index("s") * _NC + lax.axis_index("c")
    base = wid * _ROWS_W
    pltpu.sync_copy(idx_hbm.at[wid], idx_v)

    pltpu.async_copy(table_hbm.at[idx_v.at[0]], rows0, g0)

    def body(h2, _):
      j = h2 * 2
      up1 = pltpu.async_copy(table_hbm.at[idx_v.at[j + 1]], rows1, g1)
      pltpu.make_async_copy(table_hbm.at[idx_v.at[j]], rows0, g0).wait()
      pltpu.sync_copy(rows0, out_hbm.at[j, pl.ds(base, _ROWS_W)])

      @pl.when(j + 2 < _HH)
      def _():
        pltpu.async_copy(table_hbm.at[idx_v.at[j + 2]], rows0, g0)

      up1.wait()
      pltpu.sync_copy(rows1, out_hbm.at[j + 1, pl.ds(base, _ROWS_W)])
      return 0

    lax.fori_loop(0, _HH // 2, body, 0)
    pltpu.make_async_copy(table_hbm.at[idx_v.at[_HH - 1]], rows0, g0).wait()
    pltpu.sync_copy(rows0, out_hbm.at[_HH - 1, pl.ds(base, _ROWS_W)])

  return gather_kernel


_gather = _make_kernel()


def kernel(label_ids, table):
  ids = label_ids.astype(jnp.int32).reshape(_NW, _ROWS_W, HIST)
  idx = ids.transpose(0, 2, 1)
  lo = _gather(idx[:, :_HH], table)
  hi = _gather(idx[:, _HH:], table)
  out = jnp.concatenate([lo, hi], axis=0)
  return out.transpose(1, 0, 2)


# speedup vs baseline: 1.0410x; 1.0410x over previous
import functools

import jax
import jax.numpy as jnp
from jax import lax
from jax.experimental import pallas as pl
from jax.experimental.pallas import tpu as pltpu
from jax.experimental.pallas import tpu_sc as plsc

NUM_LABELS = 1000000
EMBED_DIM = 32
BATCH = 16384
HIST = 50

_NC = 2
_NS = 16
_NW = _NC * _NS
_ROWS_W = BATCH // _NW


def _make_kernel():
  mesh = plsc.VectorSubcoreMesh(core_axis_name="c", subcore_axis_name="s")

  @functools.partial(
      pl.kernel,
      out_type=jax.ShapeDtypeStruct((HIST, BATCH, EMBED_DIM), jnp.float32),
      mesh=mesh,
      compiler_params=pltpu.CompilerParams(use_tc_tiling_on_sc=False),
      scratch_types=[
          pltpu.VMEM((HIST, _ROWS_W), jnp.int32),
          pltpu.VMEM((_ROWS_W, EMBED_DIM), jnp.float32),
          pltpu.VMEM((_ROWS_W, EMBED_DIM), jnp.float32),
          pltpu.SemaphoreType.DMA,
          pltpu.SemaphoreType.DMA,
      ],
  )
  def gather_kernel(idx_hbm, table_hbm, out_hbm, idx_v, rows0, rows1, g0, g1):
    wid = lax.axis_index("s") * _NC + lax.axis_index("c")
    base = wid * _ROWS_W
    pltpu.sync_copy(idx_hbm.at[wid], idx_v)

    pltpu.async_copy(table_hbm.at[idx_v.at[0]], rows0, g0)

    def body(h2, _):
      j = h2 * 2
      up1 = pltpu.async_copy(table_hbm.at[idx_v.at[j + 1]], rows1, g1)
      pltpu.make_async_copy(table_hbm.at[idx_v.at[j]], rows0, g0).wait()
      pltpu.sync_copy(rows0, out_hbm.at[j, pl.ds(base, _ROWS_W)])

      @pl.when(j + 2 < HIST)
      def _():
        pltpu.async_copy(table_hbm.at[idx_v.at[j + 2]], rows0, g0)

      up1.wait()
      pltpu.sync_copy(rows1, out_hbm.at[j + 1, pl.ds(base, _ROWS_W)])
      return 0

    lax.fori_loop(0, HIST // 2, body, 0)

  return gather_kernel


_gather = _make_kernel()


def kernel(label_ids, table):
  ids = label_ids.astype(jnp.int32).reshape(_NW, _ROWS_W, HIST)
  idx = ids.transpose(0, 2, 1)
  out = _gather(idx, table)
  return out.transpose(1, 0, 2)
